# baseline (speedup 1.0000x reference)
import jax
import jax.numpy as jnp
from jax.experimental import pallas as pl
from jax.experimental.pallas import tpu as pltpu


_DN0 = (((0,), (0,)), ((), ()))


def _make_kernel(channels, tile, n_tiles, n_steps):
    def _fused_kernel(ca_ref,
                      x_hbm, e_hbm, n_hbm,
                      w1_ref,
                      b1_ref, temb_ref,
                      w2_ref, b2_ref,
                      out_ref,
                      x_buf, e_buf, n_buf,
                      xsem, esem, nsem):
        j = pl.program_id(0)
        b = j // n_tiles
        k = j % n_tiles

        def start(step, slot):
            bb = step // n_tiles
            kk = step % n_tiles
            sl = pl.ds(kk * tile, tile)
            pltpu.make_async_copy(
                x_hbm.at[bb, :, sl], x_buf.at[slot], xsem.at[slot]).start(priority=0)
            pltpu.make_async_copy(
                e_hbm.at[bb, :, sl], e_buf.at[slot], esem.at[slot]).start(priority=1)
            pltpu.make_async_copy(
                n_hbm.at[bb, :, sl], n_buf.at[slot], nsem.at[slot]).start(priority=1)

        @pl.when(j == 0)
        def _():
            start(0, 0)

        @pl.when(j + 1 < n_steps)
        def _():
            start(j + 1, (j + 1) % 2)

        slot = j % 2
        sl0 = pl.ds(0, tile)
        pltpu.make_async_copy(x_hbm.at[0, :, sl0], x_buf.at[slot], xsem.at[slot]).wait()
        pltpu.make_async_copy(e_hbm.at[0, :, sl0], e_buf.at[slot], esem.at[slot]).wait()
        pltpu.make_async_copy(n_hbm.at[0, :, sl0], n_buf.at[slot], nsem.at[slot]).wait()

        c = ca_ref[b]
        s = jnp.sqrt(jnp.maximum(1.0 - c * c, 0.0))

        x = x_buf[slot]
        e = e_buf[slot]
        nz = n_buf[slot]

        x_noisy = c * (x - e) + s * nz

        w1x = w1_ref[:channels].astype(jnp.bfloat16)
        w1n = w1_ref[channels:].astype(jnp.bfloat16)

        h = (jax.lax.dot_general(w1x, x.astype(jnp.bfloat16), _DN0,
                                 preferred_element_type=jnp.float32)
             + jax.lax.dot_general(w1n, x_noisy.astype(jnp.bfloat16), _DN0,
                                   preferred_element_type=jnp.float32))
        h = h + (b1_ref[...] + c * temb_ref[...]).reshape(-1, 1)
        h = h * jax.nn.sigmoid(h)

        out = (jax.lax.dot_general(w2_ref[...].astype(jnp.bfloat16),
                                   h.astype(jnp.bfloat16), _DN0,
                                   preferred_element_type=jnp.float32)
               + b2_ref[...].reshape(-1, 1))

        diff = nz - out
        psum = jnp.sum(diff * diff, axis=1)

        @pl.when(k == 0)
        def _():
            out_ref[0, 0] = psum

        @pl.when(k != 0)
        def _():
            out_ref[0, 0] = out_ref[0, 0] + psum

    return _fused_kernel


def _pick_tile(dhw, cap=8192):
    if dhw % 128 != 0:
        return dhw
    t = min(dhw, cap)
    while dhw % t != 0:
        t -= 128
    return t


def kernel(x, e, noise, sqrt_alpha, w1, b1, temb, w2, b2):
    B, C, D, H, W = x.shape
    DHW = D * H * W
    HID = w1.shape[1]

    T = _pick_tile(DHW)
    n_tiles = DHW // T
    n_steps = B * n_tiles

    xr = x.reshape(B, C, DHW)
    er = e.reshape(B, C, DHW)
    nr = noise.reshape(B, C, DHW)

    grid_spec = pltpu.PrefetchScalarGridSpec(
        num_scalar_prefetch=1,
        grid=(n_steps,),
        in_specs=[
            pl.BlockSpec(memory_space=pl.ANY),
            pl.BlockSpec(memory_space=pl.ANY),
            pl.BlockSpec(memory_space=pl.ANY),
            pl.BlockSpec((2 * C, HID), lambda j, ca: (0, 0)),
            pl.BlockSpec((1, HID), lambda j, ca: (0, 0)),
            pl.BlockSpec((1, HID), lambda j, ca: (0, 0)),
            pl.BlockSpec((HID, C), lambda j, ca: (0, 0)),
            pl.BlockSpec((1, C), lambda j, ca: (0, 0)),
        ],
        out_specs=pl.BlockSpec((1, 1, C), lambda j, ca: (j // n_tiles, 0, 0)),
        scratch_shapes=[
            pltpu.VMEM((2, C, T), jnp.float32),
            pltpu.VMEM((2, C, T), jnp.float32),
            pltpu.VMEM((2, C, T), jnp.float32),
            pltpu.SemaphoreType.DMA((2,)),
            pltpu.SemaphoreType.DMA((2,)),
            pltpu.SemaphoreType.DMA((2,)),
        ],
    )

    partials = pl.pallas_call(
        _make_kernel(C, T, n_tiles, n_steps),
        out_shape=jax.ShapeDtypeStruct((B, 1, C), jnp.float32),
        grid_spec=grid_spec,
        compiler_params=pltpu.CompilerParams(
            dimension_semantics=("arbitrary",),
            vmem_limit_bytes=64 * 1024 * 1024),
    )(sqrt_alpha, xr, er, nr, w1, b1, temb, w2, b2)

    return jnp.sum(partials) / (B * C * DHW)


# speedup vs baseline: 3.4570x; 3.4570x over previous
import jax
import jax.numpy as jnp
from jax.experimental import pallas as pl
from jax.experimental.pallas import tpu as pltpu


def _make_kernel(channels):
    def _fused_kernel(ca_ref,
                      x_ref, e_ref, n_ref,
                      w1_ref,
                      b1_ref, temb_ref,
                      w2_ref, b2_ref,
                      out_ref):
        b = pl.program_id(0)
        k = pl.program_id(1)

        c = ca_ref[b]
        s = jnp.sqrt(jnp.maximum(1.0 - c * c, 0.0))

        x = x_ref[0]
        e = e_ref[0]
        nz = n_ref[0]

        x_noisy = c * (x - e) + s * nz

        w1x = w1_ref[:channels].astype(jnp.bfloat16)
        w1n = w1_ref[channels:].astype(jnp.bfloat16)

        h = (jnp.dot(x.astype(jnp.bfloat16), w1x,
                     preferred_element_type=jnp.float32)
             + jnp.dot(x_noisy.astype(jnp.bfloat16), w1n,
                       preferred_element_type=jnp.float32))
        h = h + (b1_ref[...] + c * temb_ref[...])
        h = h * jax.nn.sigmoid(h)

        out = (jnp.dot(h.astype(jnp.bfloat16),
                       w2_ref[...].astype(jnp.bfloat16),
                       preferred_element_type=jnp.float32)
               + b2_ref[...])

        diff = nz - out
        psum = jnp.sum(diff * diff, axis=0)

        @pl.when(k == 0)
        def _():
            out_ref[0, 0] = jnp.zeros_like(psum)

        out_ref[0, 0] = out_ref[0, 0] + psum

    return _fused_kernel


def _pick_tile(dhw, cap=8192):
    if dhw % 8 != 0:
        return dhw
    t = min(dhw, cap)
    while dhw % t != 0:
        t -= 8
    return t


def kernel(x, e, noise, sqrt_alpha, w1, b1, temb, w2, b2):
    B, C, D, H, W = x.shape
    DHW = D * H * W
    HID = w1.shape[1]

    S = _pick_tile(DHW)
    n_tiles = DHW // S

    xt = jnp.transpose(x, (0, 2, 3, 4, 1)).reshape(B, DHW, C)
    et = jnp.transpose(e, (0, 2, 3, 4, 1)).reshape(B, DHW, C)
    nt = jnp.transpose(noise, (0, 2, 3, 4, 1)).reshape(B, DHW, C)

    grid_spec = pltpu.PrefetchScalarGridSpec(
        num_scalar_prefetch=1,
        grid=(B, n_tiles),
        in_specs=[
            pl.BlockSpec((1, S, C), lambda b, k, ca: (b, k, 0)),
            pl.BlockSpec((1, S, C), lambda b, k, ca: (b, k, 0)),
            pl.BlockSpec((1, S, C), lambda b, k, ca: (b, k, 0)),
            pl.BlockSpec((2 * C, HID), lambda b, k, ca: (0, 0)),
            pl.BlockSpec((1, HID), lambda b, k, ca: (0, 0)),
            pl.BlockSpec((1, HID), lambda b, k, ca: (0, 0)),
            pl.BlockSpec((HID, C), lambda b, k, ca: (0, 0)),
            pl.BlockSpec((1, C), lambda b, k, ca: (0, 0)),
        ],
        out_specs=pl.BlockSpec((1, 1, C), lambda b, k, ca: (b, 0, 0)),
    )

    partials = pl.pallas_call(
        _make_kernel(C),
        out_shape=jax.ShapeDtypeStruct((B, 1, C), jnp.float32),
        grid_spec=grid_spec,
        compiler_params=pltpu.CompilerParams(
            dimension_semantics=("arbitrary", "arbitrary"),
            vmem_limit_bytes=64 * 1024 * 1024),
    )(sqrt_alpha, xt, et, nt, w1, b1, temb, w2, b2)

    return jnp.sum(partials) / (B * C * DHW)
